# baseline (speedup 1.0000x reference)
import functools

import jax
import jax.numpy as jnp
from jax import lax
from jax.experimental import pallas as pl
from jax.experimental.pallas import tpu as pltpu
from jax.experimental.pallas import tpu_sc as plsc

NC = 2
NS = 16
NW = NC * NS
BW = 128
L = 16


def _body(xt_hbm, emb4_hbm, post_hbm, out_hbm, tok_v, idx4_v, rows4_v,
          xp_v, pos_v, sem):
    cid = lax.axis_index("c")
    sid = lax.axis_index("s")
    wid = sid * NC + cid

    seq = xt_hbm.shape[0]
    hid = pos_v.shape[0]
    b0 = wid * BW

    pltpu.sync_copy(post_hbm, pos_v)
    iota = lax.iota(jnp.int32, L)

    def sloop(s, carry):
        pltpu.sync_copy(xt_hbm.at[s, pl.ds(b0, BW)], tok_v)
        for g in range(BW // L):
            tv = tok_v[pl.ds(g * L, L)]
            idx4_v[pl.ds(g * L, L)] = lax.shift_right_logical(tv, 2)
        pltpu.async_copy(emb4_hbm.at[idx4_v], rows4_v, sem).wait()

        cbases = []
        for g in range(BW // L):
            tv = tok_v[pl.ds(g * L, L)]
            cbases.append(lax.shift_left(jnp.bitwise_and(tv, 3), 5))
        sv = jnp.full((L,), s, jnp.int32)
        for h in range(hid):
            hv = jnp.full((L,), h, jnp.int32)
            p = plsc.load_gather(pos_v, [hv, sv])
            for g in range(BW // L):
                vec = plsc.load_gather(rows4_v, [iota + (g * L), cbases[g] + h])
                xp_v[h, pl.ds(g * L, L)] = vec + p
        pltpu.sync_copy(xp_v, out_hbm.at[s, pl.ds(0, hid), pl.ds(b0, BW)])
        return carry

    lax.fori_loop(0, seq, sloop, 0)


def kernel(x, emb_table, pos_table):
    batch, seq_len = x.shape
    hid = emb_table.shape[1]

    xt = jnp.transpose(x)
    emb4 = emb_table.reshape(emb_table.shape[0] * hid // 128, 128)
    post = jnp.transpose(pos_table)

    call = pl.kernel(
        _body,
        out_type=jax.ShapeDtypeStruct((seq_len, hid, batch), jnp.float32),
        mesh=plsc.VectorSubcoreMesh(core_axis_name="c", subcore_axis_name="s"),
        scratch_types=[
            pltpu.VMEM((BW,), jnp.int32),
            pltpu.VMEM((BW,), jnp.int32),
            pltpu.VMEM((BW, 128), jnp.float32),
            pltpu.VMEM((hid, BW), jnp.float32),
            pltpu.VMEM((hid, seq_len), jnp.float32),
            pltpu.SemaphoreType.DMA,
        ],
        compiler_params=pltpu.CompilerParams(
            use_tc_tiling_on_sc=True, needs_layout_passes=False),
    )
    out = call(xt, emb4, post)
    return jnp.transpose(out, (2, 0, 1))


# speedup vs baseline: 1.1988x; 1.1988x over previous
import functools

import jax
import jax.numpy as jnp
from jax import lax
from jax.experimental import pallas as pl
from jax.experimental.pallas import tpu as pltpu
from jax.experimental.pallas import tpu_sc as plsc

NC = 2
NS = 16
NW = NC * NS
BW = 128
L = 16
NG = BW // L


def _body(xt_hbm, emb4_hbm, post_hbm, out_hbm, tok_v, idx4_a, idx4_b, rows_a,
          rows_b, xp_v, pos_v, psp_v, sem_a, sem_b):
    cid = lax.axis_index("c")
    sid = lax.axis_index("s")
    wid = sid * NC + cid

    seq = xt_hbm.shape[0]
    hid = pos_v.shape[0]
    b0 = wid * BW

    pltpu.sync_copy(post_hbm, pos_v)
    pltpu.sync_copy(xt_hbm.at[pl.ds(0, seq), pl.ds(b0, BW)], tok_v)

    iota = lax.iota(jnp.int32, L)
    ridx = [iota + g * L for g in range(NG)]

    def fire(s, idx_buf, rows_buf, sem):
        for g in range(NG):
            tv = tok_v[s, pl.ds(g * L, L)]
            idx_buf[pl.ds(g * L, L)] = lax.shift_right_logical(tv, 2)
        return pltpu.async_copy(emb4_hbm.at[idx_buf], rows_buf, sem)

    def compute(s, rows_buf):
        sv = jnp.full((L,), s, jnp.int32)
        for h in range(hid):
            psp_v[h] = plsc.load_gather(pos_v, [jnp.full((L,), h, jnp.int32), sv])
        for g in range(NG):
            tv = tok_v[s, pl.ds(g * L, L)]
            cidx = lax.shift_left(jnp.bitwise_and(tv, 3), 5)
            for h in range(hid):
                vec = plsc.load_gather(rows_buf, [ridx[g], cidx])
                xp_v[h, pl.ds(g * L, L)] = vec + psp_v[h]
                if h + 1 < hid:
                    cidx = cidx + 1
        pltpu.sync_copy(xp_v, out_hbm.at[s, pl.ds(0, hid), pl.ds(b0, BW)])

    cp0 = fire(0, idx4_a, rows_a, sem_a)

    def sloop(i, carry):
        s0 = 2 * i
        s1 = s0 + 1
        s2 = jnp.minimum(s0 + 2, seq - 1)
        cp_b = fire(s1, idx4_b, rows_b, sem_b)
        pltpu.make_async_copy(emb4_hbm.at[idx4_a], rows_a, sem_a).wait()
        compute(s0, rows_a)
        cp_a = fire(s2, idx4_a, rows_a, sem_a)
        pltpu.make_async_copy(emb4_hbm.at[idx4_b], rows_b, sem_b).wait()
        compute(s1, rows_b)
        return carry

    lax.fori_loop(0, seq // 2, sloop, 0)
    pltpu.make_async_copy(emb4_hbm.at[idx4_a], rows_a, sem_a).wait()


def kernel(x, emb_table, pos_table):
    batch, seq_len = x.shape
    hid = emb_table.shape[1]

    xt = jnp.transpose(x)
    emb4 = emb_table.reshape(emb_table.shape[0] * hid // 128, 128)
    post = jnp.transpose(pos_table)

    call = pl.kernel(
        _body,
        out_type=jax.ShapeDtypeStruct((seq_len, hid, batch), jnp.float32),
        mesh=plsc.VectorSubcoreMesh(core_axis_name="c", subcore_axis_name="s"),
        scratch_types=[
            pltpu.VMEM((seq_len, BW), jnp.int32),
            pltpu.VMEM((BW,), jnp.int32),
            pltpu.VMEM((BW,), jnp.int32),
            pltpu.VMEM((BW, 128), jnp.float32),
            pltpu.VMEM((BW, 128), jnp.float32),
            pltpu.VMEM((hid, BW), jnp.float32),
            pltpu.VMEM((hid, seq_len), jnp.float32),
            pltpu.VMEM((hid, L), jnp.float32),
            pltpu.SemaphoreType.DMA,
            pltpu.SemaphoreType.DMA,
        ],
        compiler_params=pltpu.CompilerParams(
            use_tc_tiling_on_sc=True, needs_layout_passes=False),
    )
    out = call(xt, emb4, post)
    return jnp.transpose(out, (2, 0, 1))


# speedup vs baseline: 1.7613x; 1.4692x over previous
import functools

import jax
import jax.numpy as jnp
from jax import lax
from jax.experimental import pallas as pl
from jax.experimental.pallas import tpu as pltpu
from jax.experimental.pallas import tpu_sc as plsc

NC = 2
NS = 16
NW = NC * NS
BW = 128
L = 16
NG = BW // L


def _body(xt_hbm, emb4_hbm, post_hbm, out_hbm, tok_v, idx4_a, idx4_b, rows_a,
          rows_b, xp_v, pos_v, sem_a, sem_b):
    cid = lax.axis_index("c")
    sid = lax.axis_index("s")
    wid = sid * NC + cid

    seq = xt_hbm.shape[0]
    hid = pos_v.shape[0]
    b0 = wid * BW

    pltpu.sync_copy(post_hbm, pos_v)
    pltpu.sync_copy(xt_hbm.at[pl.ds(0, seq), pl.ds(b0, BW)], tok_v)

    iota = lax.iota(jnp.int32, L)
    ridx = [iota + g * L for g in range(NG)]

    def fire(s, idx_buf, rows_buf, sem):
        for g in range(NG):
            tv = tok_v[s, pl.ds(g * L, L)]
            idx_buf[pl.ds(g * L, L)] = lax.shift_right_logical(tv, 2)
        return pltpu.async_copy(emb4_hbm.at[idx_buf], rows_buf, sem)

    def compute(s, rows_buf):
        sv = jnp.full((L,), s, jnp.int32)
        cbase = []
        for g in range(NG):
            tv = tok_v[s, pl.ds(g * L, L)]
            cbase.append(lax.shift_left(jnp.bitwise_and(tv, 3), 5))

        @plsc.parallel_loop(0, hid, step=1, unroll=4)
        def _h(h):
            hv = jnp.full((L,), h, jnp.int32)
            p = plsc.load_gather(pos_v, [hv, sv])
            for g in range(NG):
                vec = plsc.load_gather(rows_buf, [ridx[g], cbase[g] + hv])
                xp_v[h, pl.ds(g * L, L)] = vec + p

        pltpu.sync_copy(xp_v, out_hbm.at[s, pl.ds(0, hid), pl.ds(b0, BW)])

    cp0 = fire(0, idx4_a, rows_a, sem_a)

    def sloop(i, carry):
        s0 = 2 * i
        s1 = s0 + 1
        s2 = jnp.minimum(s0 + 2, seq - 1)
        cp_b = fire(s1, idx4_b, rows_b, sem_b)
        pltpu.make_async_copy(emb4_hbm.at[idx4_a], rows_a, sem_a).wait()
        compute(s0, rows_a)
        cp_a = fire(s2, idx4_a, rows_a, sem_a)
        pltpu.make_async_copy(emb4_hbm.at[idx4_b], rows_b, sem_b).wait()
        compute(s1, rows_b)
        return carry

    lax.fori_loop(0, seq // 2, sloop, 0)
    pltpu.make_async_copy(emb4_hbm.at[idx4_a], rows_a, sem_a).wait()


def kernel(x, emb_table, pos_table):
    batch, seq_len = x.shape
    hid = emb_table.shape[1]

    xt = jnp.transpose(x)
    emb4 = emb_table.reshape(emb_table.shape[0] * hid // 128, 128)
    post = jnp.transpose(pos_table)

    call = pl.kernel(
        _body,
        out_type=jax.ShapeDtypeStruct((seq_len, hid, batch), jnp.float32),
        mesh=plsc.VectorSubcoreMesh(core_axis_name="c", subcore_axis_name="s"),
        scratch_types=[
            pltpu.VMEM((seq_len, BW), jnp.int32),
            pltpu.VMEM((BW,), jnp.int32),
            pltpu.VMEM((BW,), jnp.int32),
            pltpu.VMEM((BW, 128), jnp.float32),
            pltpu.VMEM((BW, 128), jnp.float32),
            pltpu.VMEM((hid, BW), jnp.float32),
            pltpu.VMEM((hid, seq_len), jnp.float32),
            pltpu.SemaphoreType.DMA,
            pltpu.SemaphoreType.DMA,
        ],
        compiler_params=pltpu.CompilerParams(
            use_tc_tiling_on_sc=True, needs_layout_passes=False),
    )
    out = call(xt, emb4, post)
    return jnp.transpose(out, (2, 0, 1))
